# Initial kernel scaffold; baseline (speedup 1.0000x reference)
import jax
import jax.numpy as jnp
from jax.experimental import pallas as pl


def kernel(hidden_states, Wr, Wg, Wu, Wd):
    raise NotImplementedError("write your pallas kernel here")


# speedup vs baseline: 1.3915x; 1.3915x over previous
import functools

import jax
import jax.numpy as jnp
from jax.experimental import pallas as pl
from jax.experimental.pallas import tpu as pltpu

_T = 2048
_D = 1024
_E = 8
_K = 2
_F = 2816
_FB = 256


def _router_kernel(x_ref, wr_ref, logits_ref, rw_ref, idx_ref, w_ref, wdense_ref):
    x = x_ref[...]
    wr = wr_ref[...]
    logits = jax.lax.dot_general(
        x, wr, (((1,), (0,)), ((), ())),
        preferred_element_type=jnp.float32,
    )
    rw = jax.nn.softmax(logits, axis=-1)
    eidx = jax.lax.broadcasted_iota(jnp.int32, (_T, _E), 1)
    m1 = jnp.max(rw, axis=1, keepdims=True)
    i1 = jnp.min(jnp.where(rw >= m1, eidx, _E), axis=1, keepdims=True)
    masked = jnp.where(eidx == i1, -jnp.inf, rw)
    m2 = jnp.max(masked, axis=1, keepdims=True)
    i2 = jnp.min(jnp.where(masked >= m2, eidx, _E), axis=1, keepdims=True)
    wsum = m1 + m2
    w1 = m1 / wsum
    w2 = m2 / wsum
    logits_ref[...] = logits
    rw_ref[...] = rw
    idx_ref[...] = jnp.concatenate([i1, i2], axis=1)
    w_ref[...] = jnp.concatenate([w1, w2], axis=1)
    wdense_ref[...] = (w1 * (eidx == i1).astype(jnp.float32)
                       + w2 * (eidx == i2).astype(jnp.float32))


def _router(x, wr):
    return pl.pallas_call(
        _router_kernel,
        out_shape=(
            jax.ShapeDtypeStruct((_T, _E), jnp.float32),
            jax.ShapeDtypeStruct((_T, _E), jnp.float32),
            jax.ShapeDtypeStruct((_T, _K), jnp.int32),
            jax.ShapeDtypeStruct((_T, _K), jnp.float32),
            jax.ShapeDtypeStruct((_T, _E), jnp.float32),
        ),
    )(x, wr)


def _expert_kernel(x_ref, wdense_ref, wg_ref, wu_ref, wd_ref, out_ref):
    e = pl.program_id(0)
    f = pl.program_id(1)

    @pl.when(jnp.logical_and(e == 0, f == 0))
    def _init():
        out_ref[...] = jnp.zeros_like(out_ref)

    xb = x_ref[...].astype(jnp.bfloat16)
    g = jnp.dot(xb, wg_ref[0].astype(jnp.bfloat16),
                preferred_element_type=jnp.float32)
    u = jnp.dot(xb, wu_ref[0].astype(jnp.bfloat16),
                preferred_element_type=jnp.float32)
    eidx = jax.lax.broadcasted_iota(jnp.int32, (_T, _E), 1)
    we = jnp.sum(wdense_ref[...] * (eidx == e).astype(jnp.float32),
                 axis=1, keepdims=True)
    h = (jax.nn.silu(g) * u) * we
    out_ref[...] += jnp.dot(h.astype(jnp.bfloat16),
                            wd_ref[0].astype(jnp.bfloat16),
                            preferred_element_type=jnp.float32)


def _experts(x, wdense, wg, wu, wd):
    nf = _F // _FB
    return pl.pallas_call(
        _expert_kernel,
        grid=(_E, nf),
        in_specs=[
            pl.BlockSpec((_T, _D), lambda e, f: (0, 0)),
            pl.BlockSpec((_T, _E), lambda e, f: (0, 0)),
            pl.BlockSpec((1, _D, _FB), lambda e, f: (e, 0, f)),
            pl.BlockSpec((1, _D, _FB), lambda e, f: (e, 0, f)),
            pl.BlockSpec((1, _FB, _D), lambda e, f: (e, f, 0)),
        ],
        out_specs=pl.BlockSpec((_T, _D), lambda e, f: (0, 0)),
        out_shape=jax.ShapeDtypeStruct((_T, _D), jnp.float32),
        compiler_params=pltpu.CompilerParams(
            dimension_semantics=("arbitrary", "arbitrary")),
    )(x, wdense, wg, wu, wd)


@jax.jit
def kernel(hidden_states, Wr, Wg, Wu, Wd):
    b, s, d = hidden_states.shape
    x = hidden_states.reshape(s, d)
    logits, rw, idx, w, wdense = _router(x, Wr)
    out = _experts(x, wdense, Wg, Wu, Wd)
    return (out.reshape(b, s, d),
            logits.reshape(b, s, _E),
            idx.reshape(b, s, _K),
            w.reshape(b, s, _K),
            rw.reshape(b, s, _E))
